# Initial kernel scaffold; baseline (speedup 1.0000x reference)
import jax
import jax.numpy as jnp
from jax.experimental import pallas as pl


def kernel(emb_ent, emb_rel, W1_nb, W1_self, W2_nb, W2_self, W_dec, b_dec, edge_index, edge_type, triples, label):
    raise NotImplementedError("write your pallas kernel here")


# speedup vs baseline: 3.5534x; 3.5534x over previous
import functools

import jax
import jax.numpy as jnp
from jax import lax
from jax.experimental import pallas as pl
from jax.experimental.pallas import tpu as pltpu
from jax.experimental.pallas import tpu_sc as plsc

_N_ENTS = 10000
_N_RELS = 200
_H = 128
_E = 320000
_B = 2048

_NC = 2
_NS = 16
_NW = _NC * _NS
_CHUNK = 128
_EPW = 10112
_NCHUNK = _EPW // _CHUNK
_EPAD = _EPW * _NW
_RPAD = 10240
_RPT = _RPAD // _NS

_SLOPE = (1.0 / 8.0 + 1.0 / 3.0) / 2.0


def _segsum_body(table_hbm, src_hbm, dst_hbm, zeros_hbm, out_hbm,
                 idx_v, dsti_v, rows_v, acc_sh, sem):
    c = lax.axis_index("c")
    s = lax.axis_index("s")
    wid = s * _NC + c
    pltpu.sync_copy(zeros_hbm.at[pl.ds(s * _RPT, _RPT)],
                    acc_sh.at[pl.ds(s * _RPT, _RPT)])
    plsc.subcore_barrier()

    def body(i, carry):
        base = wid * _EPW + i * _CHUNK
        pltpu.sync_copy(src_hbm.at[pl.ds(base, _CHUNK)], idx_v)
        pltpu.sync_copy(dst_hbm.at[pl.ds(base, _CHUNK)], dsti_v)
        pltpu.async_copy(table_hbm.at[idx_v], rows_v, sem).wait()
        pltpu.sync_copy(rows_v, acc_sh.at[dsti_v], add=True)
        return carry

    lax.fori_loop(0, _NCHUNK, body, 0)
    plsc.subcore_barrier()
    pltpu.sync_copy(acc_sh.at[pl.ds(s * _RPT, _RPT)],
                    out_hbm.at[c, pl.ds(s * _RPT, _RPT)])


def _deg_body(dst_hbm, zeros_hbm, ones_hbm, deg_out,
              dsti_v, ones_v, acc_sh):
    c = lax.axis_index("c")
    s = lax.axis_index("s")
    wid = s * _NC + c
    pltpu.sync_copy(zeros_hbm.at[pl.ds(s * _RPT, _RPT)],
                    acc_sh.at[pl.ds(s * _RPT, _RPT)])
    pltpu.sync_copy(ones_hbm, ones_v)
    plsc.subcore_barrier()

    def body(i, carry):
        base = wid * _EPW + i * _CHUNK
        pltpu.sync_copy(dst_hbm.at[pl.ds(base, _CHUNK)], dsti_v)
        pltpu.sync_copy(ones_v, acc_sh.at[dsti_v], add=True)
        return carry

    lax.fori_loop(0, _NCHUNK, body, 0)
    plsc.subcore_barrier()
    pltpu.sync_copy(acc_sh.at[pl.ds(s * _RPT, _RPT)],
                    deg_out.at[c, pl.ds(s * _RPT, _RPT)])


def _pairgather_body(h_hbm, rel_hbm, subj_hbm, relix_hbm, es_out, r_out,
                     idx_v, rows_v, sem):
    c = lax.axis_index("c")
    s = lax.axis_index("s")
    wid = s * _NC + c
    base = wid * _CHUNK
    pltpu.sync_copy(subj_hbm.at[pl.ds(base, _CHUNK)], idx_v)
    pltpu.async_copy(h_hbm.at[idx_v], rows_v, sem).wait()
    pltpu.sync_copy(rows_v, es_out.at[pl.ds(base, _CHUNK)])
    pltpu.sync_copy(relix_hbm.at[pl.ds(base, _CHUNK)], idx_v)
    pltpu.async_copy(rel_hbm.at[idx_v], rows_v, sem).wait()
    pltpu.sync_copy(rows_v, r_out.at[pl.ds(base, _CHUNK)])


@functools.cache
def _sc_kernels():
    mesh = plsc.VectorSubcoreMesh(core_axis_name="c", subcore_axis_name="s")
    segsum = pl.kernel(
        _segsum_body,
        mesh=mesh,
        out_type=jax.ShapeDtypeStruct((_NC, _RPAD, _H), jnp.float32),
        scratch_types=[
            pltpu.VMEM((_CHUNK,), jnp.int32),
            pltpu.VMEM((_CHUNK,), jnp.int32),
            pltpu.VMEM((_CHUNK, _H), jnp.float32),
            pltpu.VMEM_SHARED((_RPAD, _H), jnp.float32),
            pltpu.SemaphoreType.DMA,
        ],
    )
    deg128 = pl.kernel(
        _deg_body,
        mesh=mesh,
        out_type=jax.ShapeDtypeStruct((_NC, _RPAD, _H), jnp.float32),
        scratch_types=[
            pltpu.VMEM((_CHUNK,), jnp.int32),
            pltpu.VMEM((_CHUNK, _H), jnp.float32),
            pltpu.VMEM_SHARED((_RPAD, _H), jnp.float32),
        ],
    )
    pairgather = pl.kernel(
        _pairgather_body,
        mesh=mesh,
        out_type=(jax.ShapeDtypeStruct((2 * _B, _H), jnp.float32),
                  jax.ShapeDtypeStruct((2 * _B, _H), jnp.float32)),
        scratch_types=[
            pltpu.VMEM((_CHUNK,), jnp.int32),
            pltpu.VMEM((_CHUNK, _H), jnp.float32),
            pltpu.SemaphoreType.DMA,
        ],
    )
    return segsum, deg128, pairgather


def _layer_body(hsum_ref, relsum_ref, deg_ref, h_ref, wnb_ref, wself_ref,
                o_ref):
    hs = (hsum_ref[0] + hsum_ref[1]) - (relsum_ref[0] + relsum_ref[1])
    deg = deg_ref[0, :, 0:1] + deg_ref[1, :, 0:1]
    norm = 1.0 / jnp.maximum(deg, 1.0)
    acc = (jnp.dot(hs * norm, wnb_ref[...], preferred_element_type=jnp.float32)
           + jnp.dot(h_ref[...], wself_ref[...],
                     preferred_element_type=jnp.float32))
    o_ref[...] = jnp.where(acc >= 0, acc, _SLOPE * acc)


_LBLK = 1280

_layer_specs_in = [
    pl.BlockSpec((_NC, _LBLK, _H), lambda i: (0, i, 0)),
    pl.BlockSpec((_NC, _LBLK, _H), lambda i: (0, i, 0)),
    pl.BlockSpec((_NC, _LBLK, _H), lambda i: (0, i, 0)),
    pl.BlockSpec((_LBLK, _H), lambda i: (i, 0)),
    pl.BlockSpec((_H, _H), lambda i: (0, 0)),
    pl.BlockSpec((_H, _H), lambda i: (0, 0)),
]
_layer_specs_out = pl.BlockSpec((_LBLK, _H), lambda i: (i, 0))

_layer = pl.pallas_call(
    _layer_body,
    grid=(_RPAD // _LBLK,),
    in_specs=_layer_specs_in,
    out_specs=_layer_specs_out,
    out_shape=jax.ShapeDtypeStruct((_RPAD, _H), jnp.float32),
)


_RB = 512
_EB = 1280


def _decoder_body(es_ref, r_ref, lab_ref, h2_ref, wdec_ref, bdec_ref, o_ref):
    i = pl.program_id(0)
    es = es_ref[...]
    r = r_ref[...]
    q = jnp.tanh(es + r)
    x = (jnp.dot(es, wdec_ref[0:_H, :], preferred_element_type=jnp.float32)
         + jnp.dot(r, wdec_ref[_H:2 * _H, :], preferred_element_type=jnp.float32)
         + jnp.dot(q, wdec_ref[2 * _H:3 * _H, :], preferred_element_type=jnp.float32)
         + bdec_ref[...])
    x = jnp.maximum(x, 0.0)
    lab = lab_ref[0, 0, :][:, None]

    def body(j, carry):
        m, sa, p = carry
        h2b = h2_ref[pl.ds(j * _EB, _EB), :]
        t = lax.dot_general(x, h2b, (((1,), (1,)), ((), ())),
                            preferred_element_type=jnp.float32)
        colid = j * _EB + lax.broadcasted_iota(jnp.int32, (_RB, _EB), 1)
        t = jnp.where(colid < _N_ENTS, t, -jnp.inf)
        mnew = jnp.maximum(m, jnp.max(t, axis=1, keepdims=True))
        sa = (sa * jnp.exp(m - mnew)
              + jnp.sum(jnp.exp(t - mnew), axis=1, keepdims=True))
        p = p + jnp.sum(jnp.where(colid == lab, t, 0.0), axis=1, keepdims=True)
        return (mnew, sa, p)

    m0 = jnp.full((_RB, 1), -jnp.inf, dtype=jnp.float32)
    s0 = jnp.zeros((_RB, 1), dtype=jnp.float32)
    p0 = jnp.zeros((_RB, 1), dtype=jnp.float32)
    m, sa, p = lax.fori_loop(0, _RPAD // _EB, body, (m0, s0, p0))
    logz = jnp.log(sa) + m
    blocksum = (jnp.sum(logz - p) / (2.0 * _B)).reshape(1, 1)

    @pl.when(i == 0)
    def _():
        o_ref[...] = jnp.zeros((1, 1), jnp.float32)

    o_ref[...] += blocksum


_dec_specs_in = [
    pl.BlockSpec((_RB, _H), lambda i: (i, 0)),
    pl.BlockSpec((_RB, _H), lambda i: (i, 0)),
    pl.BlockSpec((1, 1, _RB), lambda i: (i, 0, 0)),
    pl.BlockSpec((_RPAD, _H), lambda i: (0, 0)),
    pl.BlockSpec((3 * _H, _H), lambda i: (0, 0)),
    pl.BlockSpec((1, _H), lambda i: (0, 0)),
]
_dec_specs_out = pl.BlockSpec((1, 1), lambda i: (0, 0))

_decoder = pl.pallas_call(
    _decoder_body,
    grid=(2 * _B // _RB,),
    in_specs=_dec_specs_in,
    out_specs=_dec_specs_out,
    out_shape=jax.ShapeDtypeStruct((1, 1), jnp.float32),
)


def kernel(emb_ent, emb_rel, W1_nb, W1_self, W2_nb, W2_self, W_dec, b_dec,
           edge_index, edge_type, triples, label):
    npad = _EPAD - _E
    src_p = jnp.concatenate([edge_index[0].astype(jnp.int32),
                             jnp.zeros((npad,), jnp.int32)])
    dst_p = jnp.concatenate([edge_index[1].astype(jnp.int32),
                             jnp.full((npad,), _N_ENTS, jnp.int32)])
    et_p = jnp.concatenate([edge_type.astype(jnp.int32),
                            jnp.zeros((npad,), jnp.int32)])
    zeros128 = jnp.zeros((_RPAD, _H), jnp.float32)
    ones128 = jnp.ones((_CHUNK, _H), jnp.float32)
    emb_pad = jnp.concatenate(
        [emb_ent, jnp.zeros((_RPAD - _N_ENTS, _H), jnp.float32)])

    segsum, deg128, pairgather = _sc_kernels()
    relsum = segsum(emb_rel, et_p, dst_p, zeros128)
    deg = deg128(dst_p, zeros128, ones128)
    hsum1 = segsum(emb_pad, src_p, dst_p, zeros128)
    h1 = _layer(hsum1, relsum, deg, emb_pad, W1_nb, W1_self)
    hsum2 = segsum(h1, src_p, dst_p, zeros128)
    h2 = _layer(hsum2, relsum, deg, h1, W2_nb, W2_self)

    subj = jnp.concatenate([triples[:, 0], triples[:, 2]]).astype(jnp.int32)
    relix = jnp.concatenate([triples[:, 1],
                             triples[:, 1] + _N_RELS]).astype(jnp.int32)
    e_s, r = pairgather(h2, emb_rel, subj, relix)

    lab3 = label.astype(jnp.int32).reshape(2 * _B // _RB, 1, _RB)
    out = _decoder(e_s, r, lab3, h2, W_dec, b_dec.reshape(1, _H))
    return out[0, 0]
